# Initial kernel scaffold; baseline (speedup 1.0000x reference)
import jax
import jax.numpy as jnp
from jax.experimental import pallas as pl


def kernel(h, edge_index):
    raise NotImplementedError("write your pallas kernel here")


# speedup vs baseline: 1.7132x; 1.7132x over previous
import functools

import jax
import jax.numpy as jnp
from jax import lax
from jax.experimental import pallas as pl
from jax.experimental.pallas import tpu as pltpu
from jax.experimental.pallas import tpu_sc as plsc

N_NODES_ = 10000
N_EDGES_ = 320000
D_ = 128

_NC = 2
_NS = 16
_NW = _NC * _NS

_CHUNK = 128
_CHUNKS_PER_W = 80
_EPW = _CHUNK * _CHUNKS_PER_W
_E_PAD = _EPW * _NW


def _sc_body(h_hbm, src_hbm, dst_hbm, out_hbm,
             sidx, didx, srows, drows, scbuf, trans,
             ssem0, ssem1, dsem0, dsem1):
  wid = lax.axis_index("s") * _NC + lax.axis_index("c")
  base = wid * _EPW

  ssem = (ssem0, ssem1)
  dsem = (dsem0, dsem1)

  pltpu.sync_copy(src_hbm.at[pl.ds(base, _EPW)], sidx)
  pltpu.sync_copy(dst_hbm.at[pl.ds(base, _EPW)], didx)

  def start_gathers(b, cur):
    iv = pl.ds(cur * _CHUNK, _CHUNK)
    pltpu.async_copy(h_hbm.at[sidx.at[iv]], srows.at[b], ssem[b])
    pltpu.async_copy(h_hbm.at[didx.at[iv]], drows.at[b], dsem[b])

  def wait_gathers(b, cur):
    iv = pl.ds(cur * _CHUNK, _CHUNK)
    pltpu.make_async_copy(h_hbm.at[sidx.at[iv]], srows.at[b], ssem[b]).wait()
    pltpu.make_async_copy(h_hbm.at[didx.at[iv]], drows.at[b], dsem[b]).wait()

  for b in range(2):
    start_gathers(b, b)

  lane = lax.iota(jnp.int32, 16)

  if True:
    def compute_chunk(b, cur):
      sr = srows.at[b]
      dr = drows.at[b]

      def group_body(g, _):
        red = jnp.zeros((16,), jnp.float32)
        for e in range(16):
          eg = g * 16 + e
          acc = sr[eg, pl.ds(0, 16)] * dr[eg, pl.ds(0, 16)]
          for d in range(1, 8):
            acc = acc + sr[eg, pl.ds(d * 16, 16)] * dr[eg, pl.ds(d * 16, 16)]
          s = jnp.sum(acc)
          red = jnp.where(lane == e, s, red)
        scbuf[pl.ds(cur * _CHUNK + g * 16, 16)] = red
        return 0

      lax.fori_loop(0, _CHUNK // 16, group_body, 0)

    def loop_body(i, _):
      for b in range(2):
        cur = 2 * i + b
        wait_gathers(b, cur)
        compute_chunk(b, cur)
        nxt = cur + 2

        @pl.when(nxt < _CHUNKS_PER_W)
        def _():
          start_gathers(b, nxt)
      return 0

    lax.fori_loop(0, _CHUNKS_PER_W // 2, loop_body, 0)

  pltpu.sync_copy(scbuf, out_hbm.at[pl.ds(base, _EPW)])


@jax.jit
def _dot_predictor(h, src, dst):
  mesh = plsc.VectorSubcoreMesh(core_axis_name="c", subcore_axis_name="s")
  f = pl.kernel(
      _sc_body,
      out_type=jax.ShapeDtypeStruct((_E_PAD,), jnp.float32),
      mesh=mesh,
      compiler_params=pltpu.CompilerParams(needs_layout_passes=False),
      scratch_types=[
          pltpu.VMEM((_EPW,), jnp.int32),
          pltpu.VMEM((_EPW,), jnp.int32),
          pltpu.VMEM((2, _CHUNK, D_), jnp.float32),
          pltpu.VMEM((2, _CHUNK, D_), jnp.float32),
          pltpu.VMEM((_EPW,), jnp.float32),
          pltpu.VMEM((256,), jnp.float32),
          pltpu.SemaphoreType.DMA,
          pltpu.SemaphoreType.DMA,
          pltpu.SemaphoreType.DMA,
          pltpu.SemaphoreType.DMA,
      ],
  )
  return f(h, src, dst)[:N_EDGES_]


def kernel(h, edge_index):
  src = edge_index[0].astype(jnp.int32)
  dst = edge_index[1].astype(jnp.int32)
  pad = _E_PAD - N_EDGES_
  src = jnp.pad(src, (0, pad))
  dst = jnp.pad(dst, (0, pad))
  return _dot_predictor(h, src, dst)


# speedup vs baseline: 10.1451x; 5.9219x over previous
import functools

import jax
import jax.numpy as jnp
from jax import lax
from jax.experimental import pallas as pl
from jax.experimental.pallas import tpu as pltpu
from jax.experimental.pallas import tpu_sc as plsc

N_NODES_ = 10000
N_EDGES_ = 320000
D_ = 128

_NC = 2
_NS = 16
_NW = _NC * _NS

_CHUNK = 64
_CHUNKS_PER_W = 160
_EPW = _CHUNK * _CHUNKS_PER_W


def _sc_body(h_hbm, ei_hbm, out_hbm,
             sidx, didx, srows, drows, scb, h_sh,
             ssem0, ssem1, dsem0, dsem1,
             is0, is1, id0, id1, ws0, ws1):
  wid = lax.axis_index("s") * _NC + lax.axis_index("c")
  base = wid * _EPW
  nchunks = jnp.minimum(_CHUNKS_PER_W, (N_EDGES_ - base) // _CHUNK)

  ssem = (ssem0, ssem1)
  dsem = (dsem0, dsem1)
  isem = (is0, is1)
  idsem = (id0, id1)
  wsem = (ws0, ws1)

  @pl.when(lax.axis_index("s") == 0)
  def _():
    pltpu.sync_copy(h_hbm, h_sh)

  def idx_copies(b, cur):
    iv = pl.ds(base + cur * _CHUNK, _CHUNK)
    return (pltpu.make_async_copy(ei_hbm.at[0, iv], sidx.at[b], isem[b]),
            pltpu.make_async_copy(ei_hbm.at[1, iv], didx.at[b], idsem[b]))

  def start_idx(b, cur):
    for c in idx_copies(b, cur):
      c.start()

  def wait_idx(b, cur):
    for c in idx_copies(b, cur):
      c.wait()

  def gather_copies(b):
    return (pltpu.make_async_copy(h_sh.at[sidx.at[b]], srows.at[b], ssem[b]),
            pltpu.make_async_copy(h_sh.at[didx.at[b]], drows.at[b], dsem[b]))

  def start_gathers(b):
    for c in gather_copies(b):
      c.start()

  def wait_gathers(b):
    for c in gather_copies(b):
      c.wait()

  for b in range(2):
    start_idx(b, b)

  plsc.subcore_barrier()

  wait_idx(0, 0)
  start_gathers(0)

  lane = lax.iota(jnp.int32, 16)

  def compute_chunk(b, cur):
    sr = srows.at[b]
    dr = drows.at[b]

    def group_body(g, _):
      rowv = g * 16 + lane

      def dblk_body(dblk, carry):
        accs = list(carry)
        colv = (lane + dblk * 16) & 127
        for d2 in range(16):
          sv = plsc.load_gather(sr, [rowv, colv])
          dv = plsc.load_gather(dr, [rowv, colv])
          accs[d2 % 4] = accs[d2 % 4] + sv * dv
          if d2 < 15:
            colv = (colv + 1) & 127
        return tuple(accs)

      z = jnp.zeros((16,), jnp.float32)
      a0, a1, a2, a3 = lax.fori_loop(0, D_ // 16, dblk_body, (z, z, z, z))
      scb[b, pl.ds(g * 16, 16)] = (a0 + a1) + (a2 + a3)
      return 0

    lax.fori_loop(0, _CHUNK // 16, group_body, 0)

  def score_copy(b, cur):
    return pltpu.make_async_copy(
        scb.at[b], out_hbm.at[pl.ds(base + cur * _CHUNK, _CHUNK)], wsem[b])

  def loop_body(i, _):
    for b in range(2):
      cur = 2 * i + b

      @pl.when(cur + 1 < nchunks)
      def _():
        wait_idx(1 - b, cur + 1)
        start_gathers(1 - b)

      wait_gathers(b)

      @pl.when(cur + 2 < nchunks)
      def _():
        start_idx(b, cur + 2)

      @pl.when(cur >= 2)
      def _():
        score_copy(b, cur - 2).wait()

      compute_chunk(b, cur)
      score_copy(b, cur).start()
    return 0

  lax.fori_loop(0, nchunks // 2, loop_body, 0)

  for b in range(2):
    score_copy(b, nchunks - 2 + b).wait()


@jax.jit
def _dot_predictor(h, edge_index):
  mesh = plsc.VectorSubcoreMesh(core_axis_name="c", subcore_axis_name="s")
  f = pl.kernel(
      _sc_body,
      out_type=jax.ShapeDtypeStruct((N_EDGES_,), jnp.float32),
      mesh=mesh,
      compiler_params=pltpu.CompilerParams(needs_layout_passes=False),
      scratch_types=[
          pltpu.VMEM((2, _CHUNK), jnp.int32),
          pltpu.VMEM((2, _CHUNK), jnp.int32),
          pltpu.VMEM((2, _CHUNK, D_), jnp.float32),
          pltpu.VMEM((2, _CHUNK, D_), jnp.float32),
          pltpu.VMEM((2, _CHUNK), jnp.float32),
          pltpu.VMEM_SHARED((N_NODES_, D_), jnp.float32),
          pltpu.SemaphoreType.DMA,
          pltpu.SemaphoreType.DMA,
          pltpu.SemaphoreType.DMA,
          pltpu.SemaphoreType.DMA,
          pltpu.SemaphoreType.DMA,
          pltpu.SemaphoreType.DMA,
          pltpu.SemaphoreType.DMA,
          pltpu.SemaphoreType.DMA,
          pltpu.SemaphoreType.DMA,
          pltpu.SemaphoreType.DMA,
      ],
  )
  return f(h, edge_index)


def kernel(h, edge_index):
  return _dot_predictor(h, edge_index.astype(jnp.int32))
